# baseline (speedup 1.0000x reference)
import functools

import jax
import jax.numpy as jnp
from jax import lax
from jax.experimental import pallas as pl
from jax.experimental.pallas import tpu as pltpu
from jax.experimental.pallas import tpu_sc as plsc

N_NODES = 50000
N_EDGES = 800000
H_FEATS = 150
F = 16
NW = 32
EB = 128
KCH = 200
EPAD = NW * KCH * EB
NPAD = 53248
RPS = NPAD // 16
BM = 4096
TC_GRID = 13


def _sc_edge_aggregate(table, srcb, dstb, width):
    mesh = plsc.VectorSubcoreMesh(core_axis_name="c", subcore_axis_name="s")

    @functools.partial(
        pl.kernel,
        mesh=mesh,
        out_type=jax.ShapeDtypeStruct((2, NPAD, width), jnp.float32),
        scratch_types=[
            pltpu.VMEM((KCH, EB), jnp.int32),
            pltpu.VMEM((KCH, EB), jnp.int32),
            [pltpu.VMEM((EB, width), jnp.float32) for _ in range(4)],
            pltpu.VMEM_SHARED((NPAD, width), jnp.float32),
            [pltpu.SemaphoreType.DMA for _ in range(4)],
            [pltpu.SemaphoreType.DMA for _ in range(4)],
        ],
        compiler_params=pltpu.CompilerParams(use_tc_tiling_on_sc=False),
    )
    def k(table_h, src_h, dst_h, out_h, srcv, dstv, rows, aggs, gsem, ssem):
        cid = lax.axis_index("c")
        sid = lax.axis_index("s")
        wid = sid * 2 + cid

        for i in range(EB):
            rows[0][i, :] = jnp.zeros((width,), jnp.float32)
        for z in range(RPS // EB):
            pltpu.sync_copy(rows[0], aggs.at[pl.ds(sid * RPS + z * EB, EB)])
        plsc.subcore_barrier()

        pltpu.sync_copy(src_h.at[wid], srcv)
        pltpu.sync_copy(dst_h.at[wid], dstv)

        for b in range(4):
            pltpu.async_copy(table_h.at[srcv.at[b]], rows[b], gsem[b])

        def body(jq, carry):
            j = jq * 4
            for b in range(4):
                jb = j + b
                pltpu.make_async_copy(table_h.at[srcv.at[jb]], rows[b],
                                      gsem[b]).wait()
                pltpu.async_copy(rows[b], aggs.at[dstv.at[jb]], ssem[b],
                                 add=True)
            for b in range(4):
                jb = j + b

                @pl.when(jb + 4 < KCH)
                def _():
                    pltpu.make_async_copy(rows[b], aggs.at[dstv.at[jb]],
                                          ssem[b]).wait()
                    pltpu.async_copy(table_h.at[srcv.at[jb + 4]], rows[b],
                                     gsem[b])
            return carry

        lax.fori_loop(0, KCH // 4, body, 0)
        for b in range(4):
            pltpu.make_async_copy(rows[b], aggs.at[dstv.at[KCH - 4 + b]],
                                  ssem[b]).wait()
        plsc.subcore_barrier()

        pltpu.sync_copy(aggs.at[pl.ds(sid * RPS, RPS)],
                        out_h.at[cid, pl.ds(sid * RPS, RPS)])

    return k(table, srcb, dstb)


def _tc_layer1(x, aggp, w1s, w1n4, b1r, w2s, w2n, b2r):
    def body(x_ref, agg_ref, w1s_ref, w1n_ref, b1_ref, w2s_ref, w2n_ref,
             b2_ref, p_ref, q_ref):
        a = agg_ref[0] + agg_ref[1]
        deg = jnp.clip(a[:, 3:4], 1.0, None)
        hn = a / deg
        h = (jnp.dot(x_ref[...], w1s_ref[...], preferred_element_type=jnp.float32)
             + jnp.dot(hn, w1n_ref[...], preferred_element_type=jnp.float32)
             + b1_ref[...])
        h = jnp.maximum(h, 0.0)
        p_ref[...] = jnp.dot(h, w2n_ref[...], preferred_element_type=jnp.float32)
        q_ref[...] = (jnp.dot(h, w2s_ref[...], preferred_element_type=jnp.float32)
                      + b2_ref[...])

    return pl.pallas_call(
        body,
        grid=(TC_GRID,),
        in_specs=[
            pl.BlockSpec((BM, 3), lambda i: (i, 0)),
            pl.BlockSpec((2, BM, F), lambda i: (0, i, 0)),
            pl.BlockSpec((3, H_FEATS), lambda i: (0, 0)),
            pl.BlockSpec((F, H_FEATS), lambda i: (0, 0)),
            pl.BlockSpec((1, H_FEATS), lambda i: (0, 0)),
            pl.BlockSpec((H_FEATS, F), lambda i: (0, 0)),
            pl.BlockSpec((H_FEATS, F), lambda i: (0, 0)),
            pl.BlockSpec((1, F), lambda i: (0, 0)),
        ],
        out_specs=[pl.BlockSpec((BM, F), lambda i: (i, 0)),
                   pl.BlockSpec((BM, F), lambda i: (i, 0))],
        out_shape=[jax.ShapeDtypeStruct((N_NODES, F), jnp.float32),
                   jax.ShapeDtypeStruct((N_NODES, F), jnp.float32)],
    )(x, aggp, w1s, w1n4, b1r, w2s, w2n, b2r)


def _tc_out(q, agg2p, agg1p):
    def body(q_ref, a2_ref, a1_ref, out_ref):
        a2 = a2_ref[0] + a2_ref[1]
        a1 = a1_ref[0] + a1_ref[1]
        deg = jnp.clip(a1[:, 3:4], 1.0, None)
        out_ref[...] = q_ref[...] + a2 / deg

    return pl.pallas_call(
        body,
        grid=(TC_GRID,),
        in_specs=[
            pl.BlockSpec((BM, F), lambda i: (i, 0)),
            pl.BlockSpec((2, BM, F), lambda i: (0, i, 0)),
            pl.BlockSpec((2, BM, F), lambda i: (0, i, 0)),
        ],
        out_specs=pl.BlockSpec((BM, F), lambda i: (i, 0)),
        out_shape=jax.ShapeDtypeStruct((N_NODES, F), jnp.float32),
    )(q, agg2p, agg1p)


def kernel(in_feat, edge_index, W1_self, W1_neigh, b1, W2_self, W2_neigh, b2):
    ei = edge_index.astype(jnp.int32)
    pad = EPAD - N_EDGES
    srcb = jnp.concatenate(
        [ei[0], jnp.zeros((pad,), jnp.int32)]).reshape(NW, KCH, EB)
    dstb = jnp.concatenate(
        [ei[1], jnp.full((pad,), N_NODES, jnp.int32)]).reshape(NW, KCH, EB)
    xp = jnp.concatenate(
        [in_feat, jnp.ones((N_NODES, 1), jnp.float32),
         jnp.zeros((N_NODES, F - 4), jnp.float32)], axis=1)
    w1n4 = jnp.concatenate(
        [W1_neigh, jnp.zeros((F - 3, H_FEATS), jnp.float32)], axis=0)

    agg1 = _sc_edge_aggregate(xp, srcb, dstb, F)
    p, q = _tc_layer1(in_feat, agg1, W1_self, w1n4, b1.reshape(1, H_FEATS),
                      W2_self, W2_neigh, b2.reshape(1, F))
    agg2 = _sc_edge_aggregate(p, srcb, dstb, F)
    return _tc_out(q, agg2, agg1)


# speedup vs baseline: 1.1771x; 1.1771x over previous
import functools

import jax
import jax.numpy as jnp
from jax import lax
from jax.experimental import pallas as pl
from jax.experimental.pallas import tpu as pltpu
from jax.experimental.pallas import tpu_sc as plsc

N_NODES = 50000
N_EDGES = 800000
H_FEATS = 150
F = 16
NW = 32
EB = 128
KCH = 200
EPAD = NW * KCH * EB
NPAD = 53248
RPS = NPAD // 16
BM8 = 512
TC_GRID = 13


def _sc_edge_aggregate(table, srcb, dstb, width):
    mesh = plsc.VectorSubcoreMesh(core_axis_name="c", subcore_axis_name="s")

    @functools.partial(
        pl.kernel,
        mesh=mesh,
        out_type=jax.ShapeDtypeStruct((2, NPAD, width), jnp.float32),
        scratch_types=[
            pltpu.VMEM((KCH, EB), jnp.int32),
            pltpu.VMEM((KCH, EB), jnp.int32),
            [pltpu.VMEM((EB, width), jnp.float32) for _ in range(2)],
            pltpu.VMEM_SHARED((NPAD, width), jnp.float32),
            [pltpu.SemaphoreType.DMA for _ in range(2)],
        ],
        compiler_params=pltpu.CompilerParams(use_tc_tiling_on_sc=False),
    )
    def k(table_h, src_h, dst_h, out_h, srcv, dstv, rows, aggs, gsem):
        cid = lax.axis_index("c")
        sid = lax.axis_index("s")
        wid = sid * 2 + cid

        for i in range(EB):
            rows[0][i, :] = jnp.zeros((width,), jnp.float32)
        for z in range(RPS // EB):
            pltpu.sync_copy(rows[0], aggs.at[pl.ds(sid * RPS + z * EB, EB)])
        plsc.subcore_barrier()

        pltpu.sync_copy(src_h.at[wid], srcv)
        pltpu.sync_copy(dst_h.at[wid], dstv)

        pltpu.async_copy(table_h.at[srcv.at[0]], rows[0], gsem[0])

        def body(jh, carry):
            j = jh * 2
            pltpu.make_async_copy(table_h.at[srcv.at[j]], rows[0],
                                  gsem[0]).wait()
            pltpu.async_copy(table_h.at[srcv.at[j + 1]], rows[1], gsem[1])
            pltpu.sync_copy(rows[0], aggs.at[dstv.at[j]], add=True)

            @pl.when(j + 2 < KCH)
            def _():
                pltpu.async_copy(table_h.at[srcv.at[j + 2]], rows[0], gsem[0])

            pltpu.make_async_copy(table_h.at[srcv.at[j + 1]], rows[1],
                                  gsem[1]).wait()
            pltpu.sync_copy(rows[1], aggs.at[dstv.at[j + 1]], add=True)
            return carry

        lax.fori_loop(0, KCH // 2, body, 0)
        plsc.subcore_barrier()

        pltpu.sync_copy(aggs.at[pl.ds(sid * RPS, RPS)],
                        out_h.at[cid, pl.ds(sid * RPS, RPS)])

    return k(table, srcb, dstb)


def _tc_layer1(x24, a1, w1s_big, w1n_big, sdeg, b1_big, w2s_big, w2n_big,
               b2_big):
    def body(x_ref, a_ref, w1s_ref, w1n_ref, sdeg_ref, b1_ref, w2s_ref,
             w2n_ref, b2_ref, p_ref, q_ref):
        a = a_ref[0] + a_ref[1]
        deg = jnp.clip(
            jnp.dot(a, sdeg_ref[...], preferred_element_type=jnp.float32),
            1.0, None)
        hn = a / deg
        h = (jnp.dot(x_ref[...], w1s_ref[...], preferred_element_type=jnp.float32)
             + jnp.dot(hn, w1n_ref[...], preferred_element_type=jnp.float32)
             + b1_ref[...])
        h = jnp.maximum(h, 0.0)
        p_ref[...] = jnp.dot(h, w2n_ref[...], preferred_element_type=jnp.float32)
        q_ref[...] = (jnp.dot(h, w2s_ref[...], preferred_element_type=jnp.float32)
                      + b2_ref[...])

    hb = 8 * H_FEATS
    return pl.pallas_call(
        body,
        grid=(TC_GRID,),
        in_specs=[
            pl.BlockSpec((BM8, 24), lambda i: (i, 0)),
            pl.BlockSpec((2, BM8, 128), lambda i: (0, i, 0)),
            pl.BlockSpec((24, hb), lambda i: (0, 0)),
            pl.BlockSpec((128, hb), lambda i: (0, 0)),
            pl.BlockSpec((128, 128), lambda i: (0, 0)),
            pl.BlockSpec((1, hb), lambda i: (0, 0)),
            pl.BlockSpec((hb, 128), lambda i: (0, 0)),
            pl.BlockSpec((hb, 128), lambda i: (0, 0)),
            pl.BlockSpec((1, 128), lambda i: (0, 0)),
        ],
        out_specs=[pl.BlockSpec((BM8, 128), lambda i: (i, 0)),
                   pl.BlockSpec((BM8, 128), lambda i: (i, 0))],
        out_shape=[jax.ShapeDtypeStruct((N_NODES // 8, 128), jnp.float32),
                   jax.ShapeDtypeStruct((N_NODES // 8, 128), jnp.float32)],
    )(x24, a1, w1s_big, w1n_big, sdeg, b1_big, w2s_big, w2n_big, b2_big)


def _tc_out(q, a2, a1, sdeg):
    def body(q_ref, a2_ref, a1_ref, sdeg_ref, out_ref):
        a2b = a2_ref[0] + a2_ref[1]
        a1b = a1_ref[0] + a1_ref[1]
        deg = jnp.clip(
            jnp.dot(a1b, sdeg_ref[...], preferred_element_type=jnp.float32),
            1.0, None)
        out_ref[...] = q_ref[...] + a2b / deg

    return pl.pallas_call(
        body,
        grid=(TC_GRID,),
        in_specs=[
            pl.BlockSpec((BM8, 128), lambda i: (i, 0)),
            pl.BlockSpec((2, BM8, 128), lambda i: (0, i, 0)),
            pl.BlockSpec((2, BM8, 128), lambda i: (0, i, 0)),
            pl.BlockSpec((128, 128), lambda i: (0, 0)),
        ],
        out_specs=pl.BlockSpec((BM8, 128), lambda i: (i, 0)),
        out_shape=jax.ShapeDtypeStruct((N_NODES // 8, 128), jnp.float32),
    )(q, a2, a1, sdeg)


def kernel(in_feat, edge_index, W1_self, W1_neigh, b1, W2_self, W2_neigh, b2):
    ei = edge_index.astype(jnp.int32)
    pad = EPAD - N_EDGES
    srcb = jnp.concatenate(
        [ei[0], jnp.zeros((pad,), jnp.int32)]).reshape(NW, KCH, EB)
    dstb = jnp.concatenate(
        [ei[1], jnp.full((pad,), N_NODES, jnp.int32)]).reshape(NW, KCH, EB)
    xp = jnp.concatenate(
        [in_feat, jnp.ones((N_NODES, 1), jnp.float32),
         jnp.zeros((N_NODES, F - 4), jnp.float32)], axis=1)

    eye8 = jnp.eye(8, dtype=jnp.float32)
    sdeg = jnp.kron(eye8, jnp.zeros((F, F), jnp.float32).at[3, :].set(1.0))
    w1n16 = jnp.concatenate(
        [W1_neigh, jnp.zeros((F - 3, H_FEATS), jnp.float32)], axis=0)
    w1s_big = jnp.kron(eye8, W1_self)
    w1n_big = jnp.kron(eye8, w1n16)
    w2s_big = jnp.kron(eye8, W2_self)
    w2n_big = jnp.kron(eye8, W2_neigh)
    b1_big = jnp.tile(b1, 8).reshape(1, 8 * H_FEATS)
    b2_big = jnp.tile(b2, 8).reshape(1, 128)
    x24 = in_feat.reshape(N_NODES // 8, 24)

    agg1 = _sc_edge_aggregate(xp, srcb, dstb, F)
    a1 = agg1.reshape(2, NPAD // 8, 128)
    p128, q128 = _tc_layer1(x24, a1, w1s_big, w1n_big, sdeg, b1_big,
                            w2s_big, w2n_big, b2_big)
    p = p128.reshape(N_NODES, F)
    agg2 = _sc_edge_aggregate(p, srcb, dstb, F)
    a2 = agg2.reshape(2, NPAD // 8, 128)
    out128 = _tc_out(q128, a2, a1, sdeg)
    return out128.reshape(N_NODES, F)


# speedup vs baseline: 1.1792x; 1.0018x over previous
import functools

import jax
import jax.numpy as jnp
from jax import lax
from jax.experimental import pallas as pl
from jax.experimental.pallas import tpu as pltpu
from jax.experimental.pallas import tpu_sc as plsc

N_NODES = 50000
N_EDGES = 800000
H_FEATS = 150
F = 16
NW = 32
EB = 128
KCH = 200
EPAD = NW * KCH * EB
NPAD = 53248
RPS = NPAD // 16
BM8 = 512
TC_GRID = 13


def _sc_edge_aggregate(table, srcb, dstb, width):
    mesh = plsc.VectorSubcoreMesh(core_axis_name="c", subcore_axis_name="s")

    @functools.partial(
        pl.kernel,
        mesh=mesh,
        out_type=jax.ShapeDtypeStruct((2, NPAD, width), jnp.float32),
        scratch_types=[
            pltpu.VMEM((KCH, EB), jnp.int32),
            pltpu.VMEM((KCH, EB), jnp.int32),
            [pltpu.VMEM((EB, width), jnp.float32) for _ in range(2)],
            pltpu.VMEM_SHARED((NPAD, width), jnp.float32),
            [pltpu.SemaphoreType.DMA for _ in range(2)],
        ],
        compiler_params=pltpu.CompilerParams(use_tc_tiling_on_sc=False),
    )
    def k(table_h, src_h, dst_h, out_h, srcv, dstv, rows, aggs, gsem):
        cid = lax.axis_index("c")
        sid = lax.axis_index("s")
        wid = sid * 2 + cid

        for i in range(EB):
            rows[0][i, :] = jnp.zeros((width,), jnp.float32)
        for z in range(RPS // EB):
            pltpu.sync_copy(rows[0], aggs.at[pl.ds(sid * RPS + z * EB, EB)])
        plsc.subcore_barrier()

        pltpu.sync_copy(src_h.at[wid], srcv)
        pltpu.sync_copy(dst_h.at[wid], dstv)

        pltpu.async_copy(table_h.at[srcv.at[0]], rows[0], gsem[0])

        def body(jh, carry):
            j = jh * 2
            pltpu.make_async_copy(table_h.at[srcv.at[j]], rows[0],
                                  gsem[0]).wait()
            pltpu.async_copy(table_h.at[srcv.at[j + 1]], rows[1], gsem[1])
            pltpu.sync_copy(rows[0], aggs.at[dstv.at[j]], add=True)

            @pl.when(j + 2 < KCH)
            def _():
                pltpu.async_copy(table_h.at[srcv.at[j + 2]], rows[0], gsem[0])

            pltpu.make_async_copy(table_h.at[srcv.at[j + 1]], rows[1],
                                  gsem[1]).wait()
            pltpu.sync_copy(rows[1], aggs.at[dstv.at[j + 1]], add=True)
            return carry

        lax.fori_loop(0, KCH // 2, body, 0)
        plsc.subcore_barrier()

        pltpu.sync_copy(aggs.at[pl.ds(sid * RPS, RPS)],
                        out_h.at[cid, pl.ds(sid * RPS, RPS)])

    return k(table, srcb, dstb)


def _tc_layer1(x24, a1, w1s_big, w1n_big, sdeg, b1_big, w2s_big, w2n_big,
               b2_big):
    def body(x_ref, a_ref, w1s_ref, w1n_ref, sdeg_ref, b1_ref, w2s_ref,
             w2n_ref, b2_ref, p_ref, q_ref):
        a = a_ref[0] + a_ref[1]
        deg = jnp.clip(
            jnp.dot(a, sdeg_ref[...], preferred_element_type=jnp.float32),
            1.0, None)
        hn = a / deg
        h = (jnp.dot(x_ref[...], w1s_ref[...], preferred_element_type=jnp.float32)
             + jnp.dot(hn, w1n_ref[...], preferred_element_type=jnp.float32)
             + b1_ref[...])
        h = jnp.maximum(h, 0.0)
        p_ref[...] = jnp.dot(h, w2n_ref[...], preferred_element_type=jnp.float32)
        q_ref[...] = (jnp.dot(h, w2s_ref[...], preferred_element_type=jnp.float32)
                      + b2_ref[...])

    hb = 8 * H_FEATS
    return pl.pallas_call(
        body,
        grid=(TC_GRID,),
        in_specs=[
            pl.BlockSpec((BM8, 24), lambda i: (i, 0)),
            pl.BlockSpec((2, BM8, 128), lambda i: (0, i, 0)),
            pl.BlockSpec((24, hb), lambda i: (0, 0)),
            pl.BlockSpec((128, hb), lambda i: (0, 0)),
            pl.BlockSpec((128, 128), lambda i: (0, 0)),
            pl.BlockSpec((1, hb), lambda i: (0, 0)),
            pl.BlockSpec((hb, 128), lambda i: (0, 0)),
            pl.BlockSpec((hb, 128), lambda i: (0, 0)),
            pl.BlockSpec((1, 128), lambda i: (0, 0)),
        ],
        out_specs=[pl.BlockSpec((BM8, 128), lambda i: (i, 0)),
                   pl.BlockSpec((BM8, 128), lambda i: (i, 0))],
        out_shape=[jax.ShapeDtypeStruct((N_NODES // 8, 128), jnp.float32),
                   jax.ShapeDtypeStruct((N_NODES // 8, 128), jnp.float32)],
    )(x24, a1, w1s_big, w1n_big, sdeg, b1_big, w2s_big, w2n_big, b2_big)


def _tc_out(q, a2, a1, sdeg):
    def body(q_ref, a2_ref, a1_ref, sdeg_ref, out_ref):
        a2b = a2_ref[0] + a2_ref[1]
        a1b = a1_ref[0] + a1_ref[1]
        deg = jnp.clip(
            jnp.dot(a1b, sdeg_ref[...], preferred_element_type=jnp.float32),
            1.0, None)
        out_ref[...] = q_ref[...] + a2b / deg

    return pl.pallas_call(
        body,
        grid=(TC_GRID,),
        in_specs=[
            pl.BlockSpec((BM8, 128), lambda i: (i, 0)),
            pl.BlockSpec((2, BM8, 128), lambda i: (0, i, 0)),
            pl.BlockSpec((2, BM8, 128), lambda i: (0, i, 0)),
            pl.BlockSpec((128, 128), lambda i: (0, 0)),
        ],
        out_specs=pl.BlockSpec((BM8, 128), lambda i: (i, 0)),
        out_shape=jax.ShapeDtypeStruct((N_NODES // 8, 128), jnp.float32),
    )(q, a2, a1, sdeg)


def kernel(in_feat, edge_index, W1_self, W1_neigh, b1, W2_self, W2_neigh, b2):
    ei = edge_index.astype(jnp.int32)
    pad = EPAD - N_EDGES
    srcb = jnp.concatenate(
        [ei[0], jnp.zeros((pad,), jnp.int32)]).reshape(NW, KCH, EB)
    dst_pad = N_NODES + jnp.arange(pad, dtype=jnp.int32) % (NPAD - N_NODES)
    dstb = jnp.concatenate([ei[1], dst_pad]).reshape(NW, KCH, EB)
    xp = jnp.concatenate(
        [in_feat, jnp.ones((N_NODES, 1), jnp.float32),
         jnp.zeros((N_NODES, F - 4), jnp.float32)], axis=1)

    eye8 = jnp.eye(8, dtype=jnp.float32)
    sdeg = jnp.kron(eye8, jnp.zeros((F, F), jnp.float32).at[3, :].set(1.0))
    w1n16 = jnp.concatenate(
        [W1_neigh, jnp.zeros((F - 3, H_FEATS), jnp.float32)], axis=0)
    w1s_big = jnp.kron(eye8, W1_self)
    w1n_big = jnp.kron(eye8, w1n16)
    w2s_big = jnp.kron(eye8, W2_self)
    w2n_big = jnp.kron(eye8, W2_neigh)
    b1_big = jnp.tile(b1, 8).reshape(1, 8 * H_FEATS)
    b2_big = jnp.tile(b2, 8).reshape(1, 128)
    x24 = in_feat.reshape(N_NODES // 8, 24)

    agg1 = _sc_edge_aggregate(xp, srcb, dstb, F)
    a1 = agg1.reshape(2, NPAD // 8, 128)
    p128, q128 = _tc_layer1(x24, a1, w1s_big, w1n_big, sdeg, b1_big,
                            w2s_big, w2n_big, b2_big)
    p = p128.reshape(N_NODES, F)
    agg2 = _sc_edge_aggregate(p, srcb, dstb, F)
    a2 = agg2.reshape(2, NPAD // 8, 128)
    out128 = _tc_out(q128, a2, a1, sdeg)
    return out128.reshape(N_NODES, F)


# speedup vs baseline: 1.6330x; 1.3848x over previous
import functools

import jax
import jax.numpy as jnp
from jax import lax
from jax.experimental import pallas as pl
from jax.experimental.pallas import tpu as pltpu
from jax.experimental.pallas import tpu_sc as plsc

N_NODES = 50000
N_EDGES = 800000
H_FEATS = 150
F = 16
NW = 32
EB = 128
KCH = 196
EPAD = NW * KCH * EB
NPAD = 53248
RPS = NPAD // 16
BM8 = 512
TC_GRID = 13


def _sc_edge_aggregate(table, srcb, dstb, width):
    mesh = plsc.VectorSubcoreMesh(core_axis_name="c", subcore_axis_name="s")

    @functools.partial(
        pl.kernel,
        mesh=mesh,
        out_type=jax.ShapeDtypeStruct((2, NPAD, width), jnp.float32),
        scratch_types=[
            pltpu.VMEM((KCH, EB), jnp.int32),
            pltpu.VMEM((KCH, EB), jnp.int32),
            [pltpu.VMEM((EB, width), jnp.float32) for _ in range(2)],
            pltpu.VMEM_SHARED((NPAD, width), jnp.float32),
            [pltpu.SemaphoreType.DMA for _ in range(2)],
        ],
        compiler_params=pltpu.CompilerParams(use_tc_tiling_on_sc=False),
    )
    def k(table_h, src_h, dst_h, out_h, srcv, dstv, rows, aggs, gsem):
        cid = lax.axis_index("c")
        sid = lax.axis_index("s")
        wid = sid * 2 + cid

        for i in range(EB):
            rows[0][i, :] = jnp.zeros((width,), jnp.float32)
        for z in range(RPS // EB):
            pltpu.sync_copy(rows[0], aggs.at[pl.ds(sid * RPS + z * EB, EB)])
        plsc.subcore_barrier()

        pltpu.sync_copy(src_h.at[wid], srcv)
        pltpu.sync_copy(dst_h.at[wid], dstv)

        pltpu.async_copy(table_h.at[srcv.at[0]], rows[0], gsem[0])

        def body(jh, carry):
            j = jh * 2
            pltpu.make_async_copy(table_h.at[srcv.at[j]], rows[0],
                                  gsem[0]).wait()
            pltpu.async_copy(table_h.at[srcv.at[j + 1]], rows[1], gsem[1])
            pltpu.sync_copy(rows[0], aggs.at[dstv.at[j]], add=True)

            @pl.when(j + 2 < KCH)
            def _():
                pltpu.async_copy(table_h.at[srcv.at[j + 2]], rows[0], gsem[0])

            pltpu.make_async_copy(table_h.at[srcv.at[j + 1]], rows[1],
                                  gsem[1]).wait()
            pltpu.sync_copy(rows[1], aggs.at[dstv.at[j + 1]], add=True)
            return carry

        lax.fori_loop(0, KCH // 2, body, 0)
        plsc.subcore_barrier()

        pltpu.sync_copy(aggs.at[pl.ds(sid * RPS, RPS)],
                        out_h.at[cid, pl.ds(sid * RPS, RPS)])

    return k(table, srcb, dstb)


def _tc_layer1(x24, a1, w1s_big, w1n_big, sdeg, b1_big, w2s_big, w2n_big,
               b2_big):
    def body(x_ref, a_ref, w1s_ref, w1n_ref, sdeg_ref, b1_ref, w2s_ref,
             w2n_ref, b2_ref, p_ref, q_ref):
        a = a_ref[0] + a_ref[1]
        deg = jnp.clip(
            jnp.dot(a, sdeg_ref[...], preferred_element_type=jnp.float32),
            1.0, None)
        hn = a / deg
        h = (jnp.dot(x_ref[...], w1s_ref[...], preferred_element_type=jnp.float32)
             + jnp.dot(hn, w1n_ref[...], preferred_element_type=jnp.float32)
             + b1_ref[...])
        h = jnp.maximum(h, 0.0)
        p_ref[...] = jnp.dot(h, w2n_ref[...], preferred_element_type=jnp.float32)
        q_ref[...] = (jnp.dot(h, w2s_ref[...], preferred_element_type=jnp.float32)
                      + b2_ref[...])

    hb = 8 * H_FEATS
    return pl.pallas_call(
        body,
        grid=(TC_GRID,),
        in_specs=[
            pl.BlockSpec((BM8, 24), lambda i: (i, 0)),
            pl.BlockSpec((2, BM8, 128), lambda i: (0, i, 0)),
            pl.BlockSpec((24, hb), lambda i: (0, 0)),
            pl.BlockSpec((128, hb), lambda i: (0, 0)),
            pl.BlockSpec((128, 128), lambda i: (0, 0)),
            pl.BlockSpec((1, hb), lambda i: (0, 0)),
            pl.BlockSpec((hb, 128), lambda i: (0, 0)),
            pl.BlockSpec((hb, 128), lambda i: (0, 0)),
            pl.BlockSpec((1, 128), lambda i: (0, 0)),
        ],
        out_specs=[pl.BlockSpec((BM8, 128), lambda i: (i, 0)),
                   pl.BlockSpec((BM8, 128), lambda i: (i, 0))],
        out_shape=[jax.ShapeDtypeStruct((N_NODES // 8, 128), jnp.float32),
                   jax.ShapeDtypeStruct((N_NODES // 8, 128), jnp.float32)],
    )(x24, a1, w1s_big, w1n_big, sdeg, b1_big, w2s_big, w2n_big, b2_big)


def _tc_out(q, a2, a1, sdeg):
    def body(q_ref, a2_ref, a1_ref, sdeg_ref, out_ref):
        a2b = a2_ref[0] + a2_ref[1]
        a1b = a1_ref[0] + a1_ref[1]
        deg = jnp.clip(
            jnp.dot(a1b, sdeg_ref[...], preferred_element_type=jnp.float32),
            1.0, None)
        out_ref[...] = q_ref[...] + a2b / deg

    return pl.pallas_call(
        body,
        grid=(TC_GRID,),
        in_specs=[
            pl.BlockSpec((BM8, 128), lambda i: (i, 0)),
            pl.BlockSpec((2, BM8, 128), lambda i: (0, i, 0)),
            pl.BlockSpec((2, BM8, 128), lambda i: (0, i, 0)),
            pl.BlockSpec((128, 128), lambda i: (0, 0)),
        ],
        out_specs=pl.BlockSpec((BM8, 128), lambda i: (i, 0)),
        out_shape=jax.ShapeDtypeStruct((N_NODES // 8, 128), jnp.float32),
    )(q, a2, a1, sdeg)


def kernel(in_feat, edge_index, W1_self, W1_neigh, b1, W2_self, W2_neigh, b2):
    ei = edge_index.astype(jnp.int32)
    pad = EPAD - N_EDGES
    srcb = jnp.concatenate(
        [ei[0], jnp.zeros((pad,), jnp.int32)]).reshape(NW, KCH, EB)
    dst_pad = N_NODES + jnp.arange(pad, dtype=jnp.int32) % (NPAD - N_NODES)
    dstb = jnp.concatenate([ei[1], dst_pad]).reshape(NW, KCH, EB)
    xp = jnp.concatenate(
        [in_feat, jnp.ones((N_NODES, 1), jnp.float32),
         jnp.zeros((N_NODES, F - 4), jnp.float32)], axis=1)

    eye8 = jnp.eye(8, dtype=jnp.float32)
    sdeg = jnp.kron(eye8, jnp.zeros((F, F), jnp.float32).at[3, :].set(1.0))
    w1n16 = jnp.concatenate(
        [W1_neigh, jnp.zeros((F - 3, H_FEATS), jnp.float32)], axis=0)
    w1s_big = jnp.kron(eye8, W1_self)
    w1n_big = jnp.kron(eye8, w1n16)
    w2s_big = jnp.kron(eye8, W2_self)
    w2n_big = jnp.kron(eye8, W2_neigh)
    b1_big = jnp.tile(b1, 8).reshape(1, 8 * H_FEATS)
    b2_big = jnp.tile(b2, 8).reshape(1, 128)
    x24 = in_feat.reshape(N_NODES // 8, 24)

    agg1 = _sc_edge_aggregate(xp, srcb, dstb, F)
    a1 = agg1.reshape(2, NPAD // 8, 128)
    p128, q128 = _tc_layer1(x24, a1, w1s_big, w1n_big, sdeg, b1_big,
                            w2s_big, w2n_big, b2_big)
    p = p128.reshape(N_NODES, F)
    agg2 = _sc_edge_aggregate(p, srcb, dstb, F)
    a2 = agg2.reshape(2, NPAD // 8, 128)
    out128 = _tc_out(q128, a2, a1, sdeg)
    return out128.reshape(N_NODES, F)


# speedup vs baseline: 1.7349x; 1.0624x over previous
import functools

import jax
import jax.numpy as jnp
from jax import lax
from jax.experimental import pallas as pl
from jax.experimental.pallas import tpu as pltpu
from jax.experimental.pallas import tpu_sc as plsc

N_NODES = 50000
N_EDGES = 800000
H_FEATS = 150
F = 16
NW = 32
EB = 128
KCH = 196
EPAD = NW * KCH * EB
NPAD = 53248
RPS = NPAD // 16
BM8 = 512
TC_GRID = 13


def _sc_edge_aggregate(table, edges, width):
    mesh = plsc.VectorSubcoreMesh(core_axis_name="c", subcore_axis_name="s")

    @functools.partial(
        pl.kernel,
        mesh=mesh,
        out_type=jax.ShapeDtypeStruct((2, NPAD, width), jnp.float32),
        scratch_types=[
            pltpu.VMEM((KCH, EB), jnp.int32),
            pltpu.VMEM((KCH, EB), jnp.int32),
            [pltpu.VMEM((EB, width), jnp.float32) for _ in range(2)],
            pltpu.VMEM_SHARED((NPAD, width), jnp.float32),
            [pltpu.SemaphoreType.DMA for _ in range(2)],
        ],
        compiler_params=pltpu.CompilerParams(use_tc_tiling_on_sc=False),
    )
    def k(table_h, edges_h, out_h, srcv, dstv, rows, aggs, gsem):
        cid = lax.axis_index("c")
        sid = lax.axis_index("s")
        wid = sid * 2 + cid

        for i in range(EB):
            rows[0][i, :] = jnp.zeros((width,), jnp.float32)
        for z in range(RPS // EB):
            pltpu.sync_copy(rows[0], aggs.at[pl.ds(sid * RPS + z * EB, EB)])
        plsc.subcore_barrier()

        pltpu.sync_copy(edges_h.at[0, wid], srcv)
        pltpu.sync_copy(edges_h.at[1, wid], dstv)

        pltpu.async_copy(table_h.at[srcv.at[0]], rows[0], gsem[0])

        def body(jh, carry):
            j = jh * 2
            pltpu.make_async_copy(table_h.at[srcv.at[j]], rows[0],
                                  gsem[0]).wait()
            pltpu.async_copy(table_h.at[srcv.at[j + 1]], rows[1], gsem[1])
            pltpu.sync_copy(rows[0], aggs.at[dstv.at[j]], add=True)

            @pl.when(j + 2 < KCH)
            def _():
                pltpu.async_copy(table_h.at[srcv.at[j + 2]], rows[0], gsem[0])

            pltpu.make_async_copy(table_h.at[srcv.at[j + 1]], rows[1],
                                  gsem[1]).wait()
            pltpu.sync_copy(rows[1], aggs.at[dstv.at[j + 1]], add=True)
            return carry

        lax.fori_loop(0, KCH // 2, body, 0)
        plsc.subcore_barrier()

        pltpu.sync_copy(aggs.at[pl.ds(sid * RPS, RPS)],
                        out_h.at[cid, pl.ds(sid * RPS, RPS)])

    return k(table, edges)


def _tc_layer1(x24, a1, w1s_big, w1n_big, sdeg, b1_big, w2s_big, w2n_big,
               b2_big):
    def body(x_ref, a_ref, w1s_ref, w1n_ref, sdeg_ref, b1_ref, w2s_ref,
             w2n_ref, b2_ref, p_ref, q_ref):
        a = a_ref[0] + a_ref[1]
        deg = jnp.clip(
            jnp.dot(a, sdeg_ref[...], preferred_element_type=jnp.float32),
            1.0, None)
        hn = a / deg
        h = (jnp.dot(x_ref[...], w1s_ref[...], preferred_element_type=jnp.float32)
             + jnp.dot(hn, w1n_ref[...], preferred_element_type=jnp.float32)
             + b1_ref[...])
        h = jnp.maximum(h, 0.0)
        p_ref[...] = jnp.dot(h, w2n_ref[...], preferred_element_type=jnp.float32)
        q_ref[...] = (jnp.dot(h, w2s_ref[...], preferred_element_type=jnp.float32)
                      + b2_ref[...])

    hb = 8 * H_FEATS
    return pl.pallas_call(
        body,
        grid=(TC_GRID,),
        in_specs=[
            pl.BlockSpec((BM8, 24), lambda i: (i, 0)),
            pl.BlockSpec((2, BM8, 128), lambda i: (0, i, 0)),
            pl.BlockSpec((24, hb), lambda i: (0, 0)),
            pl.BlockSpec((128, hb), lambda i: (0, 0)),
            pl.BlockSpec((128, 128), lambda i: (0, 0)),
            pl.BlockSpec((1, hb), lambda i: (0, 0)),
            pl.BlockSpec((hb, 128), lambda i: (0, 0)),
            pl.BlockSpec((hb, 128), lambda i: (0, 0)),
            pl.BlockSpec((1, 128), lambda i: (0, 0)),
        ],
        out_specs=[pl.BlockSpec((BM8, 128), lambda i: (i, 0)),
                   pl.BlockSpec((BM8, 128), lambda i: (i, 0))],
        out_shape=[jax.ShapeDtypeStruct((N_NODES // 8, 128), jnp.float32),
                   jax.ShapeDtypeStruct((N_NODES // 8, 128), jnp.float32)],
    )(x24, a1, w1s_big, w1n_big, sdeg, b1_big, w2s_big, w2n_big, b2_big)


def _tc_out(q, a2, a1, sdeg):
    def body(q_ref, a2_ref, a1_ref, sdeg_ref, out_ref):
        a2b = a2_ref[0] + a2_ref[1]
        a1b = a1_ref[0] + a1_ref[1]
        deg = jnp.clip(
            jnp.dot(a1b, sdeg_ref[...], preferred_element_type=jnp.float32),
            1.0, None)
        out_ref[...] = q_ref[...] + a2b / deg

    return pl.pallas_call(
        body,
        grid=(TC_GRID,),
        in_specs=[
            pl.BlockSpec((BM8, 128), lambda i: (i, 0)),
            pl.BlockSpec((2, BM8, 128), lambda i: (0, i, 0)),
            pl.BlockSpec((2, BM8, 128), lambda i: (0, i, 0)),
            pl.BlockSpec((128, 128), lambda i: (0, 0)),
        ],
        out_specs=pl.BlockSpec((BM8, 128), lambda i: (i, 0)),
        out_shape=jax.ShapeDtypeStruct((N_NODES // 8, 128), jnp.float32),
    )(q, a2, a1, sdeg)


def kernel(in_feat, edge_index, W1_self, W1_neigh, b1, W2_self, W2_neigh, b2):
    ei = edge_index.astype(jnp.int32)
    pad = EPAD - N_EDGES
    pad_vals = jnp.stack(
        [jnp.zeros((pad,), jnp.int32),
         N_NODES + jnp.arange(pad, dtype=jnp.int32) % (NPAD - N_NODES)])
    edges = jnp.concatenate([ei, pad_vals], axis=1).reshape(2, NW, KCH, EB)
    xp = jnp.concatenate(
        [in_feat, jnp.ones((N_NODES, 1), jnp.float32),
         jnp.zeros((N_NODES, F - 4), jnp.float32)], axis=1)

    eye8 = jnp.eye(8, dtype=jnp.float32)
    sdeg = jnp.kron(eye8, jnp.zeros((F, F), jnp.float32).at[3, :].set(1.0))
    w1n16 = jnp.concatenate(
        [W1_neigh, jnp.zeros((F - 3, H_FEATS), jnp.float32)], axis=0)
    w1s_big = jnp.kron(eye8, W1_self)
    w1n_big = jnp.kron(eye8, w1n16)
    w2s_big = jnp.kron(eye8, W2_self)
    w2n_big = jnp.kron(eye8, W2_neigh)
    b1_big = jnp.tile(b1, 8).reshape(1, 8 * H_FEATS)
    b2_big = jnp.tile(b2, 8).reshape(1, 128)
    x24 = in_feat.reshape(N_NODES // 8, 24)

    agg1 = _sc_edge_aggregate(xp, edges, F)
    a1 = agg1.reshape(2, NPAD // 8, 128)
    p128, q128 = _tc_layer1(x24, a1, w1s_big, w1n_big, sdeg, b1_big,
                            w2s_big, w2n_big, b2_big)
    p = p128.reshape(N_NODES, F)
    agg2 = _sc_edge_aggregate(p, edges, F)
    a2 = agg2.reshape(2, NPAD // 8, 128)
    out128 = _tc_out(q128, a2, a1, sdeg)
    return out128.reshape(N_NODES, F)


# speedup vs baseline: 2.1472x; 1.2377x over previous
import functools

import jax
import jax.numpy as jnp
from jax import lax
from jax.experimental import pallas as pl
from jax.experimental.pallas import tpu as pltpu
from jax.experimental.pallas import tpu_sc as plsc

N_NODES = 50000
N_EDGES = 800000
H_FEATS = 150
F = 16
NW = 32
EB = 128
KCH = 196
EPAD = NW * KCH * EB
NPAD = 53248
RPS = NPAD // 16
BM8 = 512
TC_GRID = 13


def _sc_edge_aggregate(table, edges, width):
    mesh = plsc.VectorSubcoreMesh(core_axis_name="c", subcore_axis_name="s")

    @functools.partial(
        pl.kernel,
        mesh=mesh,
        out_type=jax.ShapeDtypeStruct((2, NPAD, width), jnp.float32),
        scratch_types=[
            pltpu.VMEM((KCH, EB), jnp.int32),
            pltpu.VMEM((KCH, EB), jnp.int32),
            [pltpu.VMEM((EB, width), jnp.float32) for _ in range(4)],
            pltpu.VMEM_SHARED((NPAD, width), jnp.float32),
            [pltpu.SemaphoreType.DMA for _ in range(4)],
            [pltpu.SemaphoreType.DMA for _ in range(4)],
        ],
        compiler_params=pltpu.CompilerParams(use_tc_tiling_on_sc=False),
    )
    def k(table_h, edges_h, out_h, srcv, dstv, rows, aggs, gsem, ssem):
        cid = lax.axis_index("c")
        sid = lax.axis_index("s")
        wid = sid * 2 + cid

        for i in range(EB):
            rows[0][i, :] = jnp.zeros((width,), jnp.float32)
        for z in range(RPS // EB):
            pltpu.sync_copy(rows[0], aggs.at[pl.ds(sid * RPS + z * EB, EB)])
        plsc.subcore_barrier()

        pltpu.sync_copy(edges_h.at[0, wid], srcv)
        pltpu.sync_copy(edges_h.at[1, wid], dstv)

        for b in range(4):
            pltpu.async_copy(table_h.at[srcv.at[b]], rows[b], gsem[b])

        def body(jq, carry):
            j = jq * 4
            for b in range(4):
                jb = j + b
                pltpu.make_async_copy(table_h.at[srcv.at[jb]], rows[b],
                                      gsem[b]).wait()
                pltpu.async_copy(rows[b], aggs.at[dstv.at[jb]], ssem[b],
                                 add=True)
            for b in range(4):
                jb = j + b

                @pl.when(jb + 4 < KCH)
                def _():
                    pltpu.make_async_copy(rows[b], aggs.at[dstv.at[jb]],
                                          ssem[b]).wait()
                    pltpu.async_copy(table_h.at[srcv.at[jb + 4]], rows[b],
                                     gsem[b])
            return carry

        lax.fori_loop(0, KCH // 4, body, 0)
        for b in range(4):
            pltpu.make_async_copy(rows[b], aggs.at[dstv.at[KCH - 4 + b]],
                                  ssem[b]).wait()
        plsc.subcore_barrier()

        pltpu.sync_copy(aggs.at[pl.ds(sid * RPS, RPS)],
                        out_h.at[cid, pl.ds(sid * RPS, RPS)])

    return k(table, edges)


def _tc_layer1(x24, a1, w1s_big, w1n_big, sdeg, b1_big, w2s_big, w2n_big,
               b2_big):
    def body(x_ref, a_ref, w1s_ref, w1n_ref, sdeg_ref, b1_ref, w2s_ref,
             w2n_ref, b2_ref, p_ref, q_ref):
        a = a_ref[0] + a_ref[1]
        deg = jnp.clip(
            jnp.dot(a, sdeg_ref[...], preferred_element_type=jnp.float32),
            1.0, None)
        hn = a / deg
        h = (jnp.dot(x_ref[...], w1s_ref[...], preferred_element_type=jnp.float32)
             + jnp.dot(hn, w1n_ref[...], preferred_element_type=jnp.float32)
             + b1_ref[...])
        h = jnp.maximum(h, 0.0)
        p_ref[...] = jnp.dot(h, w2n_ref[...], preferred_element_type=jnp.float32)
        q_ref[...] = (jnp.dot(h, w2s_ref[...], preferred_element_type=jnp.float32)
                      + b2_ref[...])

    hb = 8 * H_FEATS
    return pl.pallas_call(
        body,
        grid=(TC_GRID,),
        in_specs=[
            pl.BlockSpec((BM8, 24), lambda i: (i, 0)),
            pl.BlockSpec((2, BM8, 128), lambda i: (0, i, 0)),
            pl.BlockSpec((24, hb), lambda i: (0, 0)),
            pl.BlockSpec((128, hb), lambda i: (0, 0)),
            pl.BlockSpec((128, 128), lambda i: (0, 0)),
            pl.BlockSpec((1, hb), lambda i: (0, 0)),
            pl.BlockSpec((hb, 128), lambda i: (0, 0)),
            pl.BlockSpec((hb, 128), lambda i: (0, 0)),
            pl.BlockSpec((1, 128), lambda i: (0, 0)),
        ],
        out_specs=[pl.BlockSpec((BM8, 128), lambda i: (i, 0)),
                   pl.BlockSpec((BM8, 128), lambda i: (i, 0))],
        out_shape=[jax.ShapeDtypeStruct((N_NODES // 8, 128), jnp.float32),
                   jax.ShapeDtypeStruct((N_NODES // 8, 128), jnp.float32)],
    )(x24, a1, w1s_big, w1n_big, sdeg, b1_big, w2s_big, w2n_big, b2_big)


def _tc_out(q, a2, a1, sdeg):
    def body(q_ref, a2_ref, a1_ref, sdeg_ref, out_ref):
        a2b = a2_ref[0] + a2_ref[1]
        a1b = a1_ref[0] + a1_ref[1]
        deg = jnp.clip(
            jnp.dot(a1b, sdeg_ref[...], preferred_element_type=jnp.float32),
            1.0, None)
        out_ref[...] = q_ref[...] + a2b / deg

    return pl.pallas_call(
        body,
        grid=(TC_GRID,),
        in_specs=[
            pl.BlockSpec((BM8, 128), lambda i: (i, 0)),
            pl.BlockSpec((2, BM8, 128), lambda i: (0, i, 0)),
            pl.BlockSpec((2, BM8, 128), lambda i: (0, i, 0)),
            pl.BlockSpec((128, 128), lambda i: (0, 0)),
        ],
        out_specs=pl.BlockSpec((BM8, 128), lambda i: (i, 0)),
        out_shape=jax.ShapeDtypeStruct((N_NODES // 8, 128), jnp.float32),
    )(q, a2, a1, sdeg)


def kernel(in_feat, edge_index, W1_self, W1_neigh, b1, W2_self, W2_neigh, b2):
    ei = edge_index.astype(jnp.int32)
    pad = EPAD - N_EDGES
    pad_vals = jnp.stack(
        [jnp.zeros((pad,), jnp.int32),
         N_NODES + jnp.arange(pad, dtype=jnp.int32) % (NPAD - N_NODES)])
    edges = jnp.concatenate([ei, pad_vals], axis=1).reshape(2, NW, KCH, EB)
    xp = jnp.concatenate(
        [in_feat, jnp.ones((N_NODES, 1), jnp.float32),
         jnp.zeros((N_NODES, F - 4), jnp.float32)], axis=1)

    eye8 = jnp.eye(8, dtype=jnp.float32)
    sdeg = jnp.kron(eye8, jnp.zeros((F, F), jnp.float32).at[3, :].set(1.0))
    w1n16 = jnp.concatenate(
        [W1_neigh, jnp.zeros((F - 3, H_FEATS), jnp.float32)], axis=0)
    w1s_big = jnp.kron(eye8, W1_self)
    w1n_big = jnp.kron(eye8, w1n16)
    w2s_big = jnp.kron(eye8, W2_self)
    w2n_big = jnp.kron(eye8, W2_neigh)
    b1_big = jnp.tile(b1, 8).reshape(1, 8 * H_FEATS)
    b2_big = jnp.tile(b2, 8).reshape(1, 128)
    x24 = in_feat.reshape(N_NODES // 8, 24)

    agg1 = _sc_edge_aggregate(xp, edges, F)
    a1 = agg1.reshape(2, NPAD // 8, 128)
    p128, q128 = _tc_layer1(x24, a1, w1s_big, w1n_big, sdeg, b1_big,
                            w2s_big, w2n_big, b2_big)
    p = p128.reshape(N_NODES, F)
    agg2 = _sc_edge_aggregate(p, edges, F)
    a2 = agg2.reshape(2, NPAD // 8, 128)
    out128 = _tc_out(q128, a2, a1, sdeg)
    return out128.reshape(N_NODES, F)


# speedup vs baseline: 2.3486x; 1.0938x over previous
import functools

import jax
import jax.numpy as jnp
from jax import lax
from jax.experimental import pallas as pl
from jax.experimental.pallas import tpu as pltpu
from jax.experimental.pallas import tpu_sc as plsc

N_NODES = 50000
N_EDGES = 800000
H_FEATS = 150
F = 16
NW = 32
EB = 128
KCH = 196
EPAD = NW * KCH * EB
NPAD = 53248
RPS = NPAD // 16
BM8 = 512
TC_GRID = 13


def _sc_edge_aggregate(table, edges, width):
    mesh = plsc.VectorSubcoreMesh(core_axis_name="c", subcore_axis_name="s")

    @functools.partial(
        pl.kernel,
        mesh=mesh,
        out_type=jax.ShapeDtypeStruct((2, NPAD, width), jnp.float32),
        scratch_types=[
            pltpu.VMEM((KCH, EB), jnp.int32),
            pltpu.VMEM((KCH, EB), jnp.int32),
            [pltpu.VMEM((EB, width), jnp.float32) for _ in range(7)],
            pltpu.VMEM_SHARED((NPAD, width), jnp.float32),
            [pltpu.SemaphoreType.DMA for _ in range(7)],
            [pltpu.SemaphoreType.DMA for _ in range(7)],
        ],
        compiler_params=pltpu.CompilerParams(use_tc_tiling_on_sc=False),
    )
    def k(table_h, edges_h, out_h, srcv, dstv, rows, aggs, gsem, ssem):
        cid = lax.axis_index("c")
        sid = lax.axis_index("s")
        wid = sid * 2 + cid

        for i in range(EB):
            rows[0][i, :] = jnp.zeros((width,), jnp.float32)
        for z in range(RPS // EB):
            pltpu.sync_copy(rows[0], aggs.at[pl.ds(sid * RPS + z * EB, EB)])
        plsc.subcore_barrier()

        pltpu.sync_copy(edges_h.at[0, wid], srcv)
        pltpu.sync_copy(edges_h.at[1, wid], dstv)

        for b in range(7):
            pltpu.async_copy(table_h.at[srcv.at[b]], rows[b], gsem[b])

        def body(jq, carry):
            j = jq * 7
            for b in range(7):
                jb = j + b
                pltpu.make_async_copy(table_h.at[srcv.at[jb]], rows[b],
                                      gsem[b]).wait()
                pltpu.async_copy(rows[b], aggs.at[dstv.at[jb]], ssem[b],
                                 add=True)
            for b in range(7):
                jb = j + b

                @pl.when(jb + 7 < KCH)
                def _():
                    pltpu.make_async_copy(rows[b], aggs.at[dstv.at[jb]],
                                          ssem[b]).wait()
                    pltpu.async_copy(table_h.at[srcv.at[jb + 7]], rows[b],
                                     gsem[b])
            return carry

        lax.fori_loop(0, KCH // 7, body, 0)
        for b in range(7):
            pltpu.make_async_copy(rows[b], aggs.at[dstv.at[KCH - 7 + b]],
                                  ssem[b]).wait()
        plsc.subcore_barrier()

        pltpu.sync_copy(aggs.at[pl.ds(sid * RPS, RPS)],
                        out_h.at[cid, pl.ds(sid * RPS, RPS)])

    return k(table, edges)


def _tc_layer1(x24, a1, w1s_big, w1n_big, sdeg, b1_big, w2s_big, w2n_big,
               b2_big):
    def body(x_ref, a_ref, w1s_ref, w1n_ref, sdeg_ref, b1_ref, w2s_ref,
             w2n_ref, b2_ref, p_ref, q_ref):
        a = a_ref[0] + a_ref[1]
        deg = jnp.clip(
            jnp.dot(a, sdeg_ref[...], preferred_element_type=jnp.float32),
            1.0, None)
        hn = a / deg
        h = (jnp.dot(x_ref[...], w1s_ref[...], preferred_element_type=jnp.float32)
             + jnp.dot(hn, w1n_ref[...], preferred_element_type=jnp.float32)
             + b1_ref[...])
        h = jnp.maximum(h, 0.0)
        p_ref[...] = jnp.dot(h, w2n_ref[...], preferred_element_type=jnp.float32)
        q_ref[...] = (jnp.dot(h, w2s_ref[...], preferred_element_type=jnp.float32)
                      + b2_ref[...])

    hb = 8 * H_FEATS
    return pl.pallas_call(
        body,
        grid=(TC_GRID,),
        in_specs=[
            pl.BlockSpec((BM8, 24), lambda i: (i, 0)),
            pl.BlockSpec((2, BM8, 128), lambda i: (0, i, 0)),
            pl.BlockSpec((24, hb), lambda i: (0, 0)),
            pl.BlockSpec((128, hb), lambda i: (0, 0)),
            pl.BlockSpec((128, 128), lambda i: (0, 0)),
            pl.BlockSpec((1, hb), lambda i: (0, 0)),
            pl.BlockSpec((hb, 128), lambda i: (0, 0)),
            pl.BlockSpec((hb, 128), lambda i: (0, 0)),
            pl.BlockSpec((1, 128), lambda i: (0, 0)),
        ],
        out_specs=[pl.BlockSpec((BM8, 128), lambda i: (i, 0)),
                   pl.BlockSpec((BM8, 128), lambda i: (i, 0))],
        out_shape=[jax.ShapeDtypeStruct((N_NODES // 8, 128), jnp.float32),
                   jax.ShapeDtypeStruct((N_NODES // 8, 128), jnp.float32)],
    )(x24, a1, w1s_big, w1n_big, sdeg, b1_big, w2s_big, w2n_big, b2_big)


def _tc_out(q, a2, a1, sdeg):
    def body(q_ref, a2_ref, a1_ref, sdeg_ref, out_ref):
        a2b = a2_ref[0] + a2_ref[1]
        a1b = a1_ref[0] + a1_ref[1]
        deg = jnp.clip(
            jnp.dot(a1b, sdeg_ref[...], preferred_element_type=jnp.float32),
            1.0, None)
        out_ref[...] = q_ref[...] + a2b / deg

    return pl.pallas_call(
        body,
        grid=(TC_GRID,),
        in_specs=[
            pl.BlockSpec((BM8, 128), lambda i: (i, 0)),
            pl.BlockSpec((2, BM8, 128), lambda i: (0, i, 0)),
            pl.BlockSpec((2, BM8, 128), lambda i: (0, i, 0)),
            pl.BlockSpec((128, 128), lambda i: (0, 0)),
        ],
        out_specs=pl.BlockSpec((BM8, 128), lambda i: (i, 0)),
        out_shape=jax.ShapeDtypeStruct((N_NODES // 8, 128), jnp.float32),
    )(q, a2, a1, sdeg)


def kernel(in_feat, edge_index, W1_self, W1_neigh, b1, W2_self, W2_neigh, b2):
    ei = edge_index.astype(jnp.int32)
    pad = EPAD - N_EDGES
    pad_vals = jnp.stack(
        [jnp.zeros((pad,), jnp.int32),
         N_NODES + jnp.arange(pad, dtype=jnp.int32) % (NPAD - N_NODES)])
    edges = jnp.concatenate([ei, pad_vals], axis=1).reshape(2, NW, KCH, EB)
    xp = jnp.concatenate(
        [in_feat, jnp.ones((N_NODES, 1), jnp.float32),
         jnp.zeros((N_NODES, F - 4), jnp.float32)], axis=1)

    eye8 = jnp.eye(8, dtype=jnp.float32)
    sdeg = jnp.kron(eye8, jnp.zeros((F, F), jnp.float32).at[3, :].set(1.0))
    w1n16 = jnp.concatenate(
        [W1_neigh, jnp.zeros((F - 3, H_FEATS), jnp.float32)], axis=0)
    w1s_big = jnp.kron(eye8, W1_self)
    w1n_big = jnp.kron(eye8, w1n16)
    w2s_big = jnp.kron(eye8, W2_self)
    w2n_big = jnp.kron(eye8, W2_neigh)
    b1_big = jnp.tile(b1, 8).reshape(1, 8 * H_FEATS)
    b2_big = jnp.tile(b2, 8).reshape(1, 128)
    x24 = in_feat.reshape(N_NODES // 8, 24)

    agg1 = _sc_edge_aggregate(xp, edges, F)
    a1 = agg1.reshape(2, NPAD // 8, 128)
    p128, q128 = _tc_layer1(x24, a1, w1s_big, w1n_big, sdeg, b1_big,
                            w2s_big, w2n_big, b2_big)
    p = p128.reshape(N_NODES, F)
    agg2 = _sc_edge_aggregate(p, edges, F)
    a2 = agg2.reshape(2, NPAD // 8, 128)
    out128 = _tc_out(q128, a2, a1, sdeg)
    return out128.reshape(N_NODES, F)
